# baseline (speedup 1.0000x reference)
import functools

import jax
import jax.numpy as jnp
from jax import lax
from jax.experimental import pallas as pl


def _moe_block(x_ref, gwT_ref, gb_ref, ew_ref, eb_ref, out_ref, *, E):
    xb = x_ref[...]
    logits = jnp.dot(xb, gwT_ref[...], preferred_element_type=jnp.float32)
    logits = logits + gb_ref[...]
    BN = logits.shape[0]
    eidx = lax.broadcasted_iota(jnp.int32, (BN, E), 1)
    m1 = jnp.max(logits, axis=1, keepdims=True)
    i1 = jnp.min(jnp.where(logits == m1, eidx, E), axis=1, keepdims=True)
    masked = jnp.where(eidx == i1, -jnp.inf, logits)
    m2 = jnp.max(masked, axis=1, keepdims=True)
    i2 = jnp.min(jnp.where(masked == m2, eidx, E), axis=1, keepdims=True)
    t = jnp.exp(m2 - m1)
    w1 = 1.0 / (1.0 + t)
    w2 = t * w1
    w = jnp.where(eidx == i1, w1, 0.0) + jnp.where(eidx == i2, w2, 0.0)

    acc = jnp.dot(w, eb_ref[...], preferred_element_type=jnp.float32)
    z = jnp.concatenate([(xb * w[:, e:e + 1]).astype(jnp.bfloat16)
                         for e in range(E)], axis=1)
    acc = acc + jnp.dot(z, ew_ref[...], preferred_element_type=jnp.float32)
    out_ref[...] = acc


def kernel(x, gate_w, gate_b, expert_w, expert_b):
    N, d_in = x.shape
    E, _, d_out = expert_w.shape
    BN = 512
    grid = (N // BN,)
    gwT = gate_w.T
    gb = gate_b.reshape(1, E)
    ew16 = expert_w.astype(jnp.bfloat16).reshape(E * d_in, d_out)

    return pl.pallas_call(
        functools.partial(_moe_block, E=E),
        grid=grid,
        in_specs=[
            pl.BlockSpec((BN, d_in), lambda i: (i, 0)),
            pl.BlockSpec((d_in, E), lambda i: (0, 0)),
            pl.BlockSpec((1, E), lambda i: (0, 0)),
            pl.BlockSpec((E * d_in, d_out), lambda i: (0, 0)),
            pl.BlockSpec((E, d_out), lambda i: (0, 0)),
        ],
        out_specs=pl.BlockSpec((BN, d_out), lambda i: (i, 0)),
        out_shape=jax.ShapeDtypeStruct((N, d_out), jnp.float32),
    )(x, gwT, gb, ew16, expert_b)


# speedup vs baseline: 1.0542x; 1.0542x over previous
import functools

import jax
import jax.numpy as jnp
from jax import lax
from jax.experimental import pallas as pl


def _moe_block(x_ref, gwT_ref, gb_ref, ew_ref, eb_ref, out_ref, *, E):
    xb = x_ref[...]
    logits = jnp.dot(xb, gwT_ref[...], preferred_element_type=jnp.float32)
    logits = logits + gb_ref[...]
    BN = logits.shape[0]
    eidx = lax.broadcasted_iota(jnp.int32, (BN, E), 1)
    m1 = jnp.max(logits, axis=1, keepdims=True)
    i1 = jnp.min(jnp.where(logits == m1, eidx, E), axis=1, keepdims=True)
    masked = jnp.where(eidx == i1, -jnp.inf, logits)
    m2 = jnp.max(masked, axis=1, keepdims=True)
    i2 = jnp.min(jnp.where(masked == m2, eidx, E), axis=1, keepdims=True)
    t = jnp.exp(m2 - m1)
    w1 = 1.0 / (1.0 + t)
    w2 = t * w1
    w = jnp.where(eidx == i1, w1, 0.0) + jnp.where(eidx == i2, w2, 0.0)

    acc = jnp.dot(w, eb_ref[...], preferred_element_type=jnp.float32)
    xb16 = xb.astype(jnp.bfloat16)
    w16 = w.astype(jnp.bfloat16)
    for e in range(E):
        xs = xb16 * w16[:, e:e + 1]
        acc = acc + jnp.dot(xs, ew_ref[e], preferred_element_type=jnp.float32)
    out_ref[...] = acc


def kernel(x, gate_w, gate_b, expert_w, expert_b):
    N, d_in = x.shape
    E, _, d_out = expert_w.shape
    BN = 1024
    grid = (N // BN,)
    gwT = gate_w.T
    gb = gate_b.reshape(1, E)
    ew16 = expert_w.astype(jnp.bfloat16)

    return pl.pallas_call(
        functools.partial(_moe_block, E=E),
        grid=grid,
        in_specs=[
            pl.BlockSpec((BN, d_in), lambda i: (i, 0)),
            pl.BlockSpec((d_in, E), lambda i: (0, 0)),
            pl.BlockSpec((1, E), lambda i: (0, 0)),
            pl.BlockSpec((E, d_in, d_out), lambda i: (0, 0, 0)),
            pl.BlockSpec((E, d_out), lambda i: (0, 0)),
        ],
        out_specs=pl.BlockSpec((BN, d_out), lambda i: (i, 0)),
        out_shape=jax.ShapeDtypeStruct((N, d_out), jnp.float32),
    )(x, gwT, gb, ew16, expert_b)


# speedup vs baseline: 1.2402x; 1.1765x over previous
import functools

import jax
import jax.numpy as jnp
from jax import lax
from jax.experimental import pallas as pl
from jax.experimental.pallas import tpu as pltpu


def _moe_block(x_ref, gw_ref, gb_ref, ew_ref, eb_ref, out_ref, ew16_ref, *, E):
    i = pl.program_id(0)

    @pl.when(i == 0)
    def _cast_weights():
        ew16_ref[...] = ew_ref[...].astype(jnp.bfloat16)

    xb = x_ref[...]
    logits = lax.dot_general(
        xb, gw_ref[...], (((1,), (1,)), ((), ())),
        preferred_element_type=jnp.float32)
    logits = logits + gb_ref[...]
    BN = logits.shape[0]
    eidx = lax.broadcasted_iota(jnp.int32, (BN, E), 1)
    m1 = jnp.max(logits, axis=1, keepdims=True)
    i1 = jnp.min(jnp.where(logits == m1, eidx, E), axis=1, keepdims=True)
    masked = jnp.where(eidx == i1, -jnp.inf, logits)
    m2 = jnp.max(masked, axis=1, keepdims=True)
    i2 = jnp.min(jnp.where(masked == m2, eidx, E), axis=1, keepdims=True)
    t = jnp.exp(m2 - m1)
    w1 = 1.0 / (1.0 + t)
    w2 = t * w1
    w = jnp.where(eidx == i1, w1, 0.0) + jnp.where(eidx == i2, w2, 0.0)

    acc = jnp.dot(w, eb_ref[...], preferred_element_type=jnp.float32)
    xb16 = xb.astype(jnp.bfloat16)
    for e in range(E):
        y = jnp.dot(xb16, ew16_ref[e], preferred_element_type=jnp.float32)
        acc = acc + y * w[:, e:e + 1]
    out_ref[...] = acc


def kernel(x, gate_w, gate_b, expert_w, expert_b):
    N, d_in = x.shape
    E, _, d_out = expert_w.shape
    BN = 512
    grid = (N // BN,)
    gb = gate_b.reshape(1, E)

    return pl.pallas_call(
        functools.partial(_moe_block, E=E),
        grid=grid,
        in_specs=[
            pl.BlockSpec((BN, d_in), lambda i: (i, 0)),
            pl.BlockSpec((E, d_in), lambda i: (0, 0)),
            pl.BlockSpec((1, E), lambda i: (0, 0)),
            pl.BlockSpec((E, d_in, d_out), lambda i: (0, 0, 0)),
            pl.BlockSpec((E, d_out), lambda i: (0, 0)),
        ],
        out_specs=pl.BlockSpec((BN, d_out), lambda i: (i, 0)),
        out_shape=jax.ShapeDtypeStruct((N, d_out), jnp.float32),
        scratch_shapes=[pltpu.VMEM((E, d_in, d_out), jnp.bfloat16)],
    )(x, gate_w, gb, expert_w, expert_b)


# speedup vs baseline: 1.2916x; 1.0414x over previous
import functools

import jax
import jax.numpy as jnp
from jax import lax
from jax.experimental import pallas as pl
from jax.experimental.pallas import tpu as pltpu


def _moe_block(x_ref, gw_ref, gb_ref, ew_ref, eb_ref, out_ref, ew16_ref, *, E):
    i = pl.program_id(0)

    @pl.when(i == 0)
    def _cast_weights():
        ew16_ref[...] = ew_ref[...].astype(jnp.bfloat16)

    xb = x_ref[...]
    logits = lax.dot_general(
        xb, gw_ref[...], (((1,), (1,)), ((), ())),
        preferred_element_type=jnp.float32)
    logits = logits + gb_ref[...]
    BN = logits.shape[0]
    eidx = lax.broadcasted_iota(jnp.int32, (BN, E), 1)
    m1 = jnp.max(logits, axis=1, keepdims=True)
    i1 = jnp.min(jnp.where(logits == m1, eidx, E), axis=1, keepdims=True)
    masked = jnp.where(eidx == i1, -jnp.inf, logits)
    m2 = jnp.max(masked, axis=1, keepdims=True)
    i2 = jnp.min(jnp.where(masked == m2, eidx, E), axis=1, keepdims=True)
    t = jnp.exp(m2 - m1)
    w1 = 1.0 / (1.0 + t)
    w2 = t * w1
    w = jnp.where(eidx == i1, w1, 0.0) + jnp.where(eidx == i2, w2, 0.0)

    acc = jnp.dot(w, eb_ref[...], preferred_element_type=jnp.float32)
    xb16 = xb.astype(jnp.bfloat16)
    for e in range(E):
        y = jnp.dot(xb16, ew16_ref[e], preferred_element_type=jnp.float32)
        acc = acc + y * w[:, e:e + 1]
    out_ref[...] = acc


def kernel(x, gate_w, gate_b, expert_w, expert_b):
    N, d_in = x.shape
    E, _, d_out = expert_w.shape
    BN = 1024
    grid = (N // BN,)
    gb = gate_b.reshape(1, E)

    return pl.pallas_call(
        functools.partial(_moe_block, E=E),
        grid=grid,
        in_specs=[
            pl.BlockSpec((BN, d_in), lambda i: (i, 0)),
            pl.BlockSpec((E, d_in), lambda i: (0, 0)),
            pl.BlockSpec((1, E), lambda i: (0, 0)),
            pl.BlockSpec((E, d_in, d_out), lambda i: (0, 0, 0)),
            pl.BlockSpec((E, d_out), lambda i: (0, 0)),
        ],
        out_specs=pl.BlockSpec((BN, d_out), lambda i: (i, 0)),
        out_shape=jax.ShapeDtypeStruct((N, d_out), jnp.float32),
        scratch_shapes=[pltpu.VMEM((E, d_in, d_out), jnp.bfloat16)],
    )(x, gate_w, gb, expert_w, expert_b)
